# baseline (speedup 1.0000x reference)
import functools

import jax
import jax.numpy as jnp
from jax import lax
from jax.experimental import pallas as pl
from jax.experimental.pallas import tpu as pltpu
from jax.experimental.pallas import tpu_sc as plsc

_G = 10
_NG0 = 1000
_D = 128
_EPG = 32000
_CPAD = 1024


_EPT = _EPG // 16
_SPG = _CPAD * _CPAD
_WPT = _SPG // 16
_ECH = 8000
_NCH = _EPG // _ECH


def _sc_counts_body(src_hbm, dst_hbm, out_hbm, src_v, dst_v, stripe_v):
    cid = lax.axis_index("c")
    sid = lax.axis_index("s")
    ones = jnp.ones((16,), jnp.float32)

    for gi in range(_G // 2):
        g = gi * 2 + cid

        def zfill(i, _):
            stripe_v[pl.ds(i * 16, 16)] = jnp.zeros((16,), jnp.float32)
            return 0
        lax.fori_loop(0, _WPT // 16, zfill, 0)

        rowbase = sid * 64
        for ch in range(_NCH):
            ebase = g * _EPG + ch * _ECH
            pltpu.sync_copy(src_hbm.at[pl.ds(ebase, _ECH)], src_v)
            pltpu.sync_copy(dst_hbm.at[pl.ds(ebase, _ECH)], dst_v)

            def ebody(i, _):
                s16 = src_v[pl.ds(i * 16, 16)] - g * _NG0
                d16 = dst_v[pl.ds(i * 16, 16)] - g * _NG0
                dl = d16 - rowbase
                zero16 = jnp.zeros((16,), jnp.int32)
                msk = (dl >= zero16) & (dl < 64)
                rel = jnp.where(msk, dl * _CPAD + s16, zero16)
                plsc.addupdate_scatter(stripe_v, [rel], ones, mask=msk)
                return 0
            lax.fori_loop(0, _ECH // 16, ebody, 0)

        for cc in range(4):
            dloc = rowbase + cc * 16 + lax.iota(jnp.int32, 16)
            dmsk = dloc < _NG0
            drel = jnp.where(dmsk, (dloc - rowbase) * _CPAD + dloc,
                             jnp.zeros((16,), jnp.int32))
            plsc.addupdate_scatter(stripe_v, [drel], ones, mask=dmsk)

        gbase = g * (_NG0 * _CPAD)

        @pl.when(sid < 15)
        def _():
            pltpu.sync_copy(stripe_v,
                            out_hbm.at[pl.ds(gbase + sid * _WPT, _WPT)])

        @pl.when(sid == 15)
        def _():
            nlast = (_NG0 - 15 * 64) * _CPAD
            pltpu.sync_copy(stripe_v.at[pl.ds(0, nlast)],
                            out_hbm.at[pl.ds(gbase + 15 * _WPT, nlast)])


def _build_counts(edge_index):
    sc_counts = functools.partial(
        pl.kernel,
        out_type=jax.ShapeDtypeStruct((_G * _NG0 * _CPAD,), jnp.float32),
        mesh=plsc.VectorSubcoreMesh(core_axis_name="c", subcore_axis_name="s"),
        compiler_params=pltpu.CompilerParams(needs_layout_passes=False),
        scratch_types=[
            pltpu.VMEM((_ECH,), jnp.int32),
            pltpu.VMEM((_ECH,), jnp.int32),
            pltpu.VMEM((_WPT,), jnp.float32),
        ],
    )(_sc_counts_body)
    return sc_counts(edge_index[0],
                     edge_index[1]).reshape(_G, _NG0, _CPAD)


def _make_layer(n_g, first, last):
    n_half = n_g // 2
    n_total = n_g * _G

    def body(*refs):
        if first:
            (hin_ref, c_ref, w_ref, as_ref, ad_ref, b_ref,
             hout_ref, cout_ref, sum_ref, sq_ref) = refs
        elif last:
            (hin_ref, c_ref, w_ref, as_ref, ad_ref, b_ref, bnsum_ref,
             bnsq_ref, hout_ref, sum_ref, sq_ref) = refs
        else:
            (hin_ref, c_ref, w_ref, as_ref, ad_ref, b_ref, bnsum_ref,
             bnsq_ref, hout_ref, cout_ref, sum_ref, sq_ref) = refs

        hin = hin_ref[0]
        if not first:
            tot = jnp.sum(bnsum_ref[...], axis=0)
            totsq = jnp.sum(bnsq_ref[...], axis=0)
            mu = tot / n_total
            var = totsq / n_total - mu * mu
            hin = (hin - mu) * lax.rsqrt(var + 1e-5)

        h = jnp.dot(hin, w_ref[...], preferred_element_type=jnp.float32,
                    precision=lax.Precision.HIGHEST)
        avd = jnp.dot(h, ad_ref[...], preferred_element_type=jnp.float32,
                      precision=lax.Precision.HIGHEST)
        avs_row = lax.dot_general(as_ref[...], h, (((1,), (1,)), ((), ())),
                                  preferred_element_type=jnp.float32,
                                  precision=lax.Precision.HIGHEST)
        e = avd + avs_row
        e = jnp.where(e >= 0, e, 0.2 * e)

        c = c_ref[0]
        if first:
            c = c[:, :n_g]
        m = jnp.max(e, axis=1, keepdims=True)
        wt = c * jnp.exp(e - m)
        denom = jnp.sum(wt, axis=1, keepdims=True) + 1e-16
        out = jnp.dot(wt, h, preferred_element_type=jnp.float32,
                      precision=lax.Precision.HIGHEST) / denom
        hrelu = jnp.maximum(out + b_ref[...], 0.0)
        hp = jnp.max(hrelu.reshape(n_half, 2, _D), axis=1)
        hout_ref[0] = hp
        sum_ref[0] = jnp.sum(hp, axis=0, keepdims=True)
        sq_ref[0] = jnp.sum(hp * hp, axis=0, keepdims=True)

        if not last:
            rb = jnp.max(c.reshape(n_half, 2, n_g), axis=1)
            i3 = lax.broadcasted_iota(jnp.int32, (n_g, n_half), 0)
            j3 = lax.broadcasted_iota(jnp.int32, (n_g, n_half), 1)
            p = jnp.where(i3 // 2 == j3, 1.0, 0.0).astype(jnp.bfloat16)
            s2 = jnp.dot(rb.astype(jnp.bfloat16), p,
                         preferred_element_type=jnp.float32)
            ih = lax.broadcasted_iota(jnp.int32, (n_half, n_half), 0)
            jh = lax.broadcasted_iota(jnp.int32, (n_half, n_half), 1)
            cout_ref[0] = jnp.where((s2 > 0.5) | (ih == jh), 1.0, 0.0)

    c_minor = _CPAD if first else n_g
    in_specs = [
        pl.BlockSpec((1, n_g, _D), lambda g: (g, 0, 0)),
        pl.BlockSpec((1, n_g, c_minor), lambda g: (g, 0, 0)),
        pl.BlockSpec((_D, _D), lambda g: (0, 0)),
        pl.BlockSpec((1, _D), lambda g: (0, 0)),
        pl.BlockSpec((_D, 1), lambda g: (0, 0)),
        pl.BlockSpec((1, _D), lambda g: (0, 0)),
    ]
    if not first:
        in_specs += [
            pl.BlockSpec((_G, 1, _D), lambda g: (0, 0, 0)),
            pl.BlockSpec((_G, 1, _D), lambda g: (0, 0, 0)),
        ]
    out_shapes = [jax.ShapeDtypeStruct((_G, n_half, _D), jnp.float32)]
    out_specs = [pl.BlockSpec((1, n_half, _D), lambda g: (g, 0, 0))]
    if not last:
        out_shapes.append(jax.ShapeDtypeStruct((_G, n_half, n_half),
                                               jnp.float32))
        out_specs.append(pl.BlockSpec((1, n_half, n_half),
                                      lambda g: (g, 0, 0)))
    out_shapes += [jax.ShapeDtypeStruct((_G, 1, _D), jnp.float32),
                   jax.ShapeDtypeStruct((_G, 1, _D), jnp.float32)]
    out_specs += [pl.BlockSpec((1, 1, _D), lambda g: (g, 0, 0)),
                  pl.BlockSpec((1, 1, _D), lambda g: (g, 0, 0))]

    return pl.pallas_call(
        body,
        grid=(_G,),
        in_specs=in_specs,
        out_specs=out_specs,
        out_shape=out_shapes,
    )


def _bn_final_body(hin_ref, bnsum_ref, bnsq_ref, out_ref):
    n_total = 125 * _G
    tot = jnp.sum(bnsum_ref[...], axis=0)
    totsq = jnp.sum(bnsq_ref[...], axis=0)
    mu = tot / n_total
    var = totsq / n_total - mu * mu
    out_ref[0] = (hin_ref[0] - mu) * lax.rsqrt(var + 1e-5)


def kernel(x, edge_index, W0, as0, ad0, b0, W1, as1, ad1, b1,
           W2, as2, ad2, b2):
    counts = _build_counts(edge_index)

    h = x.reshape(_G, _NG0, _D)
    params = [(W0, as0, ad0, b0), (W1, as1, ad1, b1), (W2, as2, ad2, b2)]
    bnsum = bnsq = None
    n_g = _NG0
    c = counts
    for i in range(3):
        first = i == 0
        last = i == 2
        W, a_s, a_d, b = params[i]
        args = [h, c, W, a_s.reshape(1, _D), a_d.reshape(_D, 1),
                b.reshape(1, _D)]
        if not first:
            args += [bnsum, bnsq]
        outs = _make_layer(n_g, first, last)(*args)
        if last:
            h, bnsum, bnsq = outs
        else:
            h, c, bnsum, bnsq = outs
        n_g //= 2

    out = pl.pallas_call(
        _bn_final_body,
        grid=(_G,),
        in_specs=[
            pl.BlockSpec((1, 125, _D), lambda g: (g, 0, 0)),
            pl.BlockSpec((_G, 1, _D), lambda g: (0, 0, 0)),
            pl.BlockSpec((_G, 1, _D), lambda g: (0, 0, 0)),
        ],
        out_specs=pl.BlockSpec((1, 125, _D), lambda g: (g, 0, 0)),
        out_shape=jax.ShapeDtypeStruct((_G, 125, _D), jnp.float32),
    )(h, bnsum, bnsq)
    return out.reshape(_G, 125 * _D)


# speedup vs baseline: 1.1720x; 1.1720x over previous
import functools

import jax
import jax.numpy as jnp
from jax import lax
from jax.experimental import pallas as pl
from jax.experimental.pallas import tpu as pltpu
from jax.experimental.pallas import tpu_sc as plsc

_G = 10
_NG0 = 1000
_D = 128
_EPG = 32000
_CPAD = 1024


_EPT = _EPG // 16
_SPG = _CPAD * _CPAD
_WPT = _SPG // 16
_ECH = 8000
_NCH = _EPG // _ECH


def _sc_counts_body(src_hbm, dst_hbm, out_hbm, src_v, dst_v, stripe_v):
    cid = lax.axis_index("c")
    sid = lax.axis_index("s")
    ones = jnp.ones((16,), jnp.float32)

    for gi in range(_G // 2):
        g = gi * 2 + cid

        def zfill(i, _):
            stripe_v[pl.ds(i * 16, 16)] = jnp.zeros((16,), jnp.float32)
            return 0
        lax.fori_loop(0, _WPT // 16, zfill, 0)

        rowbase = sid * 64
        for ch in range(_NCH):
            ebase = g * _EPG + ch * _ECH
            pltpu.sync_copy(src_hbm.at[pl.ds(ebase, _ECH)], src_v)
            pltpu.sync_copy(dst_hbm.at[pl.ds(ebase, _ECH)], dst_v)

            def ebody(i, _):
                s16 = src_v[pl.ds(i * 16, 16)] - g * _NG0
                d16 = dst_v[pl.ds(i * 16, 16)] - g * _NG0
                dl = d16 - rowbase
                zero16 = jnp.zeros((16,), jnp.int32)
                msk = (dl >= zero16) & (dl < 64)
                rel = jnp.where(msk, dl * _CPAD + s16, zero16)
                plsc.addupdate_scatter(stripe_v, [rel], ones, mask=msk)
                return 0
            lax.fori_loop(0, _ECH // 16, ebody, 0)

        for cc in range(4):
            dloc = rowbase + cc * 16 + lax.iota(jnp.int32, 16)
            dmsk = dloc < _NG0
            drel = jnp.where(dmsk, (dloc - rowbase) * _CPAD + dloc,
                             jnp.zeros((16,), jnp.int32))
            plsc.addupdate_scatter(stripe_v, [drel], ones, mask=dmsk)

        gbase = g * (_NG0 * _CPAD)

        @pl.when(sid < 15)
        def _():
            pltpu.sync_copy(stripe_v,
                            out_hbm.at[pl.ds(gbase + sid * _WPT, _WPT)])

        @pl.when(sid == 15)
        def _():
            nlast = (_NG0 - 15 * 64) * _CPAD
            pltpu.sync_copy(stripe_v.at[pl.ds(0, nlast)],
                            out_hbm.at[pl.ds(gbase + 15 * _WPT, nlast)])


def _build_counts(edge_index):
    sc_counts = functools.partial(
        pl.kernel,
        out_type=jax.ShapeDtypeStruct((_G * _NG0 * _CPAD,), jnp.float32),
        mesh=plsc.VectorSubcoreMesh(core_axis_name="c", subcore_axis_name="s"),
        compiler_params=pltpu.CompilerParams(needs_layout_passes=False),
        scratch_types=[
            pltpu.VMEM((_ECH,), jnp.int32),
            pltpu.VMEM((_ECH,), jnp.int32),
            pltpu.VMEM((_WPT,), jnp.float32),
        ],
    )(_sc_counts_body)
    return sc_counts(edge_index[0],
                     edge_index[1]).reshape(_G, _NG0, _CPAD)


def _dot3(a, b, dims):
    ahi = a.astype(jnp.bfloat16)
    alo = (a - ahi.astype(jnp.float32)).astype(jnp.bfloat16)
    bhi = b.astype(jnp.bfloat16)
    blo = (b - bhi.astype(jnp.float32)).astype(jnp.bfloat16)
    dg = functools.partial(lax.dot_general, dimension_numbers=dims,
                          preferred_element_type=jnp.float32)
    return dg(ahi, bhi) + (dg(ahi, blo) + dg(alo, bhi))


_MM = (((1,), (0,)), ((), ()))
_MMT = (((1,), (1,)), ((), ()))


def _make_layer(n_g, first, last):
    n_half = n_g // 2
    n_total = n_g * _G

    def body(*refs):
        if first:
            (hin_ref, c_ref, w_ref, as_ref, ad_ref, b_ref,
             hout_ref, cout_ref, sum_ref, sq_ref) = refs
        elif last:
            (hin_ref, c_ref, w_ref, as_ref, ad_ref, b_ref, bnsum_ref,
             bnsq_ref, hout_ref, sum_ref, sq_ref) = refs
        else:
            (hin_ref, c_ref, w_ref, as_ref, ad_ref, b_ref, bnsum_ref,
             bnsq_ref, hout_ref, cout_ref, sum_ref, sq_ref) = refs

        hin = hin_ref[0]
        if not first:
            tot = jnp.sum(bnsum_ref[...], axis=0)
            totsq = jnp.sum(bnsq_ref[...], axis=0)
            mu = tot / n_total
            var = totsq / n_total - mu * mu
            hin = (hin - mu) * lax.rsqrt(var + 1e-5)

        hin_bf = hin.astype(jnp.bfloat16)
        h = lax.dot_general(hin_bf, w_ref[...].astype(jnp.bfloat16), _MM,
                            preferred_element_type=jnp.float32)
        h_bf = h.astype(jnp.bfloat16)
        avd = lax.dot_general(h_bf, ad_ref[...].astype(jnp.bfloat16), _MM,
                              preferred_element_type=jnp.float32)
        avs_row = lax.dot_general(as_ref[...].astype(jnp.bfloat16), h_bf,
                                  _MMT, preferred_element_type=jnp.float32)
        e = avd + avs_row
        e = jnp.where(e >= 0, e, 0.2 * e)

        c = c_ref[0]
        if first:
            c = c[:, :n_g]
        m = jnp.max(e, axis=1, keepdims=True)
        wt = c * jnp.exp(e - m)
        denom = jnp.sum(wt, axis=1, keepdims=True) + 1e-16
        out = _dot3(wt, h, _MM) / denom
        hrelu = jnp.maximum(out + b_ref[...], 0.0)
        hp = jnp.max(hrelu.reshape(n_half, 2, _D), axis=1)
        hout_ref[0] = hp
        sum_ref[0] = jnp.sum(hp, axis=0, keepdims=True)
        sq_ref[0] = jnp.sum(hp * hp, axis=0, keepdims=True)

        if not last:
            rb = jnp.max(c.reshape(n_half, 2, n_g), axis=1)
            i3 = lax.broadcasted_iota(jnp.int32, (n_g, n_half), 0)
            j3 = lax.broadcasted_iota(jnp.int32, (n_g, n_half), 1)
            p = jnp.where(i3 // 2 == j3, 1.0, 0.0).astype(jnp.bfloat16)
            s2 = jnp.dot(rb.astype(jnp.bfloat16), p,
                         preferred_element_type=jnp.float32)
            ih = lax.broadcasted_iota(jnp.int32, (n_half, n_half), 0)
            jh = lax.broadcasted_iota(jnp.int32, (n_half, n_half), 1)
            cout_ref[0] = jnp.where((s2 > 0.5) | (ih == jh), 1.0, 0.0)

    c_minor = _CPAD if first else n_g
    in_specs = [
        pl.BlockSpec((1, n_g, _D), lambda g: (g, 0, 0)),
        pl.BlockSpec((1, n_g, c_minor), lambda g: (g, 0, 0)),
        pl.BlockSpec((_D, _D), lambda g: (0, 0)),
        pl.BlockSpec((1, _D), lambda g: (0, 0)),
        pl.BlockSpec((_D, 1), lambda g: (0, 0)),
        pl.BlockSpec((1, _D), lambda g: (0, 0)),
    ]
    if not first:
        in_specs += [
            pl.BlockSpec((_G, 1, _D), lambda g: (0, 0, 0)),
            pl.BlockSpec((_G, 1, _D), lambda g: (0, 0, 0)),
        ]
    out_shapes = [jax.ShapeDtypeStruct((_G, n_half, _D), jnp.float32)]
    out_specs = [pl.BlockSpec((1, n_half, _D), lambda g: (g, 0, 0))]
    if not last:
        out_shapes.append(jax.ShapeDtypeStruct((_G, n_half, n_half),
                                               jnp.float32))
        out_specs.append(pl.BlockSpec((1, n_half, n_half),
                                      lambda g: (g, 0, 0)))
    out_shapes += [jax.ShapeDtypeStruct((_G, 1, _D), jnp.float32),
                   jax.ShapeDtypeStruct((_G, 1, _D), jnp.float32)]
    out_specs += [pl.BlockSpec((1, 1, _D), lambda g: (g, 0, 0)),
                  pl.BlockSpec((1, 1, _D), lambda g: (g, 0, 0))]

    return pl.pallas_call(
        body,
        grid=(_G,),
        in_specs=in_specs,
        out_specs=out_specs,
        out_shape=out_shapes,
    )


def _bn_final_body(hin_ref, bnsum_ref, bnsq_ref, out_ref):
    n_total = 125 * _G
    tot = jnp.sum(bnsum_ref[...], axis=0)
    totsq = jnp.sum(bnsq_ref[...], axis=0)
    mu = tot / n_total
    var = totsq / n_total - mu * mu
    out_ref[0] = (hin_ref[0] - mu) * lax.rsqrt(var + 1e-5)


def kernel(x, edge_index, W0, as0, ad0, b0, W1, as1, ad1, b1,
           W2, as2, ad2, b2):
    counts = _build_counts(edge_index)

    h = x.reshape(_G, _NG0, _D)
    params = [(W0, as0, ad0, b0), (W1, as1, ad1, b1), (W2, as2, ad2, b2)]
    bnsum = bnsq = None
    n_g = _NG0
    c = counts
    for i in range(3):
        first = i == 0
        last = i == 2
        W, a_s, a_d, b = params[i]
        args = [h, c, W, a_s.reshape(1, _D), a_d.reshape(_D, 1),
                b.reshape(1, _D)]
        if not first:
            args += [bnsum, bnsq]
        outs = _make_layer(n_g, first, last)(*args)
        if last:
            h, bnsum, bnsq = outs
        else:
            h, c, bnsum, bnsq = outs
        n_g //= 2

    out = pl.pallas_call(
        _bn_final_body,
        grid=(_G,),
        in_specs=[
            pl.BlockSpec((1, 125, _D), lambda g: (g, 0, 0)),
            pl.BlockSpec((_G, 1, _D), lambda g: (0, 0, 0)),
            pl.BlockSpec((_G, 1, _D), lambda g: (0, 0, 0)),
        ],
        out_specs=pl.BlockSpec((1, 125, _D), lambda g: (g, 0, 0)),
        out_shape=jax.ShapeDtypeStruct((_G, 125, _D), jnp.float32),
    )(h, bnsum, bnsq)
    return out.reshape(_G, 125 * _D)


# speedup vs baseline: 1.5415x; 1.3153x over previous
import functools

import jax
import jax.numpy as jnp
from jax import lax
from jax.experimental import pallas as pl
from jax.experimental.pallas import tpu as pltpu
from jax.experimental.pallas import tpu_sc as plsc

_G = 10
_NG0 = 1000
_D = 128
_EPG = 32000
_CPAD = 1024


_EPT = _EPG // 16
_SPG = _CPAD * _CPAD
_WPT = _SPG // 16
_ECH = 8000
_NCH = _EPG // _ECH


def _sc_counts_body(src_hbm, dst_hbm, out_hbm, src_v0, src_v1, dst_v0,
                    dst_v1, stripe_v, sem):
    srcb = (src_v0, src_v1)
    dstb = (dst_v0, dst_v1)
    cid = lax.axis_index("c")
    sid = lax.axis_index("s")
    ones = jnp.ones((16,), jnp.float32)
    rowbase = sid * 64

    for gi in range(_G // 2):
        g = gi * 2 + cid

        def zfill(i, _):
            for u in range(8):
                stripe_v[pl.ds(i * 128 + u * 16, 16)] = jnp.zeros(
                    (16,), jnp.float32)
            return 0
        lax.fori_loop(0, _WPT // 128, zfill, 0)

        def fire(ch, buf):
            ebase = g * _EPG + ch * _ECH
            return (pltpu.make_async_copy(src_hbm.at[pl.ds(ebase, _ECH)],
                                          srcb[buf], sem),
                    pltpu.make_async_copy(dst_hbm.at[pl.ds(ebase, _ECH)],
                                          dstb[buf], sem))

        def start(c):
            for d in fire(*c):
                d.start()

        def drain(c):
            for d in fire(*c):
                d.wait()

        start((0, 0))
        koff = g * (_NG0 * _CPAD + _NG0) + rowbase * _CPAD

        for ch in range(_NCH):
            drain((ch, ch % 2))
            if ch + 1 < _NCH:
                start((ch + 1, (ch + 1) % 2))

            def ebody(i, _):
                for u in range(4):
                    o = i * 64 + u * 16
                    s16 = srcb[ch % 2][pl.ds(o, 16)]
                    d16 = dstb[ch % 2][pl.ds(o, 16)]
                    rel = d16 * _CPAD + s16 - koff
                    msk = (rel >= 0) & (rel < _WPT)
                    relc = jnp.where(msk, rel, 0)
                    plsc.addupdate_scatter(stripe_v, [relc], ones, mask=msk)
                return 0
            lax.fori_loop(0, _ECH // 64, ebody, 0)

        for cc in range(4):
            dloc = rowbase + cc * 16 + lax.iota(jnp.int32, 16)
            dmsk = dloc < _NG0
            drel = jnp.where(dmsk, (dloc - rowbase) * _CPAD + dloc,
                             jnp.zeros((16,), jnp.int32))
            plsc.addupdate_scatter(stripe_v, [drel], ones, mask=dmsk)

        gbase = g * (_NG0 * _CPAD)

        @pl.when(sid < 15)
        def _():
            pltpu.sync_copy(stripe_v,
                            out_hbm.at[pl.ds(gbase + sid * _WPT, _WPT)])

        @pl.when(sid == 15)
        def _():
            nlast = (_NG0 - 15 * 64) * _CPAD
            pltpu.sync_copy(stripe_v.at[pl.ds(0, nlast)],
                            out_hbm.at[pl.ds(gbase + 15 * _WPT, nlast)])


def _build_counts(edge_index):
    sc_counts = functools.partial(
        pl.kernel,
        out_type=jax.ShapeDtypeStruct((_G * _NG0 * _CPAD,), jnp.float32),
        mesh=plsc.VectorSubcoreMesh(core_axis_name="c", subcore_axis_name="s"),
        compiler_params=pltpu.CompilerParams(needs_layout_passes=False),
        scratch_types=[
            pltpu.VMEM((_ECH,), jnp.int32),
            pltpu.VMEM((_ECH,), jnp.int32),
            pltpu.VMEM((_ECH,), jnp.int32),
            pltpu.VMEM((_ECH,), jnp.int32),
            pltpu.VMEM((_WPT,), jnp.float32),
            pltpu.SemaphoreType.DMA,
        ],
    )(_sc_counts_body)
    return sc_counts(edge_index[0],
                     edge_index[1]).reshape(_G, _NG0, _CPAD)


def _dot3(a, b, dims):
    ahi = a.astype(jnp.bfloat16)
    alo = (a - ahi.astype(jnp.float32)).astype(jnp.bfloat16)
    bhi = b.astype(jnp.bfloat16)
    blo = (b - bhi.astype(jnp.float32)).astype(jnp.bfloat16)
    dg = functools.partial(lax.dot_general, dimension_numbers=dims,
                          preferred_element_type=jnp.float32)
    return dg(ahi, bhi) + (dg(ahi, blo) + dg(alo, bhi))


_MM = (((1,), (0,)), ((), ()))
_MMT = (((1,), (1,)), ((), ()))


def _make_layer(n_g, first, last):
    n_half = n_g // 2
    n_total = n_g * _G

    def body(*refs):
        if first:
            (hin_ref, c_ref, w_ref, as_ref, ad_ref, b_ref,
             hout_ref, cout_ref, sum_ref, sq_ref) = refs
        elif last:
            (hin_ref, c_ref, w_ref, as_ref, ad_ref, b_ref, bnsum_ref,
             bnsq_ref, hout_ref, sum_ref, sq_ref) = refs
        else:
            (hin_ref, c_ref, w_ref, as_ref, ad_ref, b_ref, bnsum_ref,
             bnsq_ref, hout_ref, cout_ref, sum_ref, sq_ref) = refs

        hin = hin_ref[0]
        if not first:
            tot = jnp.sum(bnsum_ref[...], axis=0)
            totsq = jnp.sum(bnsq_ref[...], axis=0)
            mu = tot / n_total
            var = totsq / n_total - mu * mu
            hin = (hin - mu) * lax.rsqrt(var + 1e-5)

        hin_bf = hin.astype(jnp.bfloat16)
        h = lax.dot_general(hin_bf, w_ref[...].astype(jnp.bfloat16), _MM,
                            preferred_element_type=jnp.float32)
        h_bf = h.astype(jnp.bfloat16)
        avd = lax.dot_general(h_bf, ad_ref[...].astype(jnp.bfloat16), _MM,
                              preferred_element_type=jnp.float32)
        avs_row = lax.dot_general(as_ref[...].astype(jnp.bfloat16), h_bf,
                                  _MMT, preferred_element_type=jnp.float32)
        e = avd + avs_row
        e = jnp.where(e >= 0, e, 0.2 * e)

        c = c_ref[0]
        if first:
            c = c[:, :n_g]
        m = jnp.max(e, axis=1, keepdims=True)
        wt = c * jnp.exp(e - m)
        denom = jnp.sum(wt, axis=1, keepdims=True) + 1e-16
        out = _dot3(wt, h, _MM) / denom
        hrelu = jnp.maximum(out + b_ref[...], 0.0)
        hp = jnp.max(hrelu.reshape(n_half, 2, _D), axis=1)
        hout_ref[0] = hp
        sum_ref[0] = jnp.sum(hp, axis=0, keepdims=True)
        sq_ref[0] = jnp.sum(hp * hp, axis=0, keepdims=True)

        if not last:
            rb = jnp.max(c.reshape(n_half, 2, n_g), axis=1)
            i3 = lax.broadcasted_iota(jnp.int32, (n_g, n_half), 0)
            j3 = lax.broadcasted_iota(jnp.int32, (n_g, n_half), 1)
            p = jnp.where(i3 // 2 == j3, 1.0, 0.0).astype(jnp.bfloat16)
            s2 = jnp.dot(rb.astype(jnp.bfloat16), p,
                         preferred_element_type=jnp.float32)
            ih = lax.broadcasted_iota(jnp.int32, (n_half, n_half), 0)
            jh = lax.broadcasted_iota(jnp.int32, (n_half, n_half), 1)
            cout_ref[0] = jnp.where((s2 > 0.5) | (ih == jh), 1.0, 0.0)

    c_minor = _CPAD if first else n_g
    in_specs = [
        pl.BlockSpec((1, n_g, _D), lambda g: (g, 0, 0)),
        pl.BlockSpec((1, n_g, c_minor), lambda g: (g, 0, 0)),
        pl.BlockSpec((_D, _D), lambda g: (0, 0)),
        pl.BlockSpec((1, _D), lambda g: (0, 0)),
        pl.BlockSpec((_D, 1), lambda g: (0, 0)),
        pl.BlockSpec((1, _D), lambda g: (0, 0)),
    ]
    if not first:
        in_specs += [
            pl.BlockSpec((_G, 1, _D), lambda g: (0, 0, 0)),
            pl.BlockSpec((_G, 1, _D), lambda g: (0, 0, 0)),
        ]
    out_shapes = [jax.ShapeDtypeStruct((_G, n_half, _D), jnp.float32)]
    out_specs = [pl.BlockSpec((1, n_half, _D), lambda g: (g, 0, 0))]
    if not last:
        out_shapes.append(jax.ShapeDtypeStruct((_G, n_half, n_half),
                                               jnp.float32))
        out_specs.append(pl.BlockSpec((1, n_half, n_half),
                                      lambda g: (g, 0, 0)))
    out_shapes += [jax.ShapeDtypeStruct((_G, 1, _D), jnp.float32),
                   jax.ShapeDtypeStruct((_G, 1, _D), jnp.float32)]
    out_specs += [pl.BlockSpec((1, 1, _D), lambda g: (g, 0, 0)),
                  pl.BlockSpec((1, 1, _D), lambda g: (g, 0, 0))]

    return pl.pallas_call(
        body,
        grid=(_G,),
        in_specs=in_specs,
        out_specs=out_specs,
        out_shape=out_shapes,
    )


def _bn_final_body(hin_ref, bnsum_ref, bnsq_ref, out_ref):
    n_total = 125 * _G
    tot = jnp.sum(bnsum_ref[...], axis=0)
    totsq = jnp.sum(bnsq_ref[...], axis=0)
    mu = tot / n_total
    var = totsq / n_total - mu * mu
    out_ref[0] = (hin_ref[0] - mu) * lax.rsqrt(var + 1e-5)


def kernel(x, edge_index, W0, as0, ad0, b0, W1, as1, ad1, b1,
           W2, as2, ad2, b2):
    counts = _build_counts(edge_index)

    h = x.reshape(_G, _NG0, _D)
    params = [(W0, as0, ad0, b0), (W1, as1, ad1, b1), (W2, as2, ad2, b2)]
    bnsum = bnsq = None
    n_g = _NG0
    c = counts
    for i in range(3):
        first = i == 0
        last = i == 2
        W, a_s, a_d, b = params[i]
        args = [h, c, W, a_s.reshape(1, _D), a_d.reshape(_D, 1),
                b.reshape(1, _D)]
        if not first:
            args += [bnsum, bnsq]
        outs = _make_layer(n_g, first, last)(*args)
        if last:
            h, bnsum, bnsq = outs
        else:
            h, c, bnsum, bnsq = outs
        n_g //= 2

    out = pl.pallas_call(
        _bn_final_body,
        grid=(_G,),
        in_specs=[
            pl.BlockSpec((1, 125, _D), lambda g: (g, 0, 0)),
            pl.BlockSpec((_G, 1, _D), lambda g: (0, 0, 0)),
            pl.BlockSpec((_G, 1, _D), lambda g: (0, 0, 0)),
        ],
        out_specs=pl.BlockSpec((1, 125, _D), lambda g: (g, 0, 0)),
        out_shape=jax.ShapeDtypeStruct((_G, 125, _D), jnp.float32),
    )(h, bnsum, bnsq)
    return out.reshape(_G, 125 * _D)
